# baseline (speedup 1.0000x reference)
import functools

import jax
import jax.numpy as jnp
from jax import lax
from jax.experimental import pallas as pl
from jax.experimental.pallas import tpu as pltpu
from jax.experimental.pallas import tpu_sc as plsc

NC = 2
NS = 16
NW = NC * NS
LANES = 16
BE = 128
NBUF = 8
PADR = 512
PACK = 128


@functools.lru_cache(maxsize=None)
def _make_deg(n_nodes, nctot):
    nch = nctot // NW
    njb = nch // NBUF
    rps = n_nodes // NS
    mesh = plsc.VectorSubcoreMesh(core_axis_name="c", subcore_axis_name="s")

    @functools.partial(
        pl.kernel,
        mesh=mesh,
        compiler_params=pltpu.CompilerParams(use_tc_tiling_on_sc=False),
        out_type=jax.ShapeDtypeStruct((NC, n_nodes, LANES), jnp.float32),
        scratch_types=[
            pltpu.VMEM((nch, BE), jnp.int32),
            pltpu.VMEM((BE, LANES), jnp.float32),
            pltpu.VMEM_SHARED((n_nodes + PADR, LANES), jnp.float32),
            pltpu.SemaphoreType.DMA((NBUF,)),
        ],
    )
    def deg_kernel(eidx_hbm, zeros_hbm, ones_hbm, out_hbm,
                   dst_v, ones_v, acc, dsem):
        c = lax.axis_index("c")
        s = lax.axis_index("s")
        wid = s * NC + c
        pltpu.sync_copy(zeros_hbm.at[pl.ds(s * rps, rps)],
                        acc.at[pl.ds(s * rps, rps)])
        pltpu.sync_copy(ones_hbm, ones_v)
        pltpu.sync_copy(eidx_hbm.at[1, pl.ds(wid * nch, nch)], dst_v)
        plsc.subcore_barrier()

        def scat(i, b):
            pltpu.async_copy(ones_v, acc.at[dst_v.at[i]], dsem.at[b],
                             add=True)

        def scat_wait(i, b):
            pltpu.make_async_copy(ones_v, acc.at[dst_v.at[i]],
                                  dsem.at[b]).wait()

        for b in range(NBUF):
            scat(b, b)

        def body(j, carry):
            i0 = j * NBUF
            for b in range(NBUF):
                scat_wait(i0 + b, b)
                scat(i0 + NBUF + b, b)
            return carry

        lax.fori_loop(0, njb - 1, body, 0)
        i0 = (njb - 1) * NBUF
        for b in range(NBUF):
            scat_wait(i0 + b, b)
        plsc.subcore_barrier()
        pltpu.sync_copy(acc.at[pl.ds(s * rps, rps)],
                        out_hbm.at[c, pl.ds(s * rps, rps)])

    return deg_kernel


def _edge_pipeline(table, src_v, dst_v, rows_v, acc, gsem, ssem, nch):
    njb = nch // NBUF

    def gather(i, b):
        pltpu.async_copy(table.at[src_v.at[i]], rows_v.at[b], gsem.at[b])

    def gather_wait(i, b):
        pltpu.make_async_copy(table.at[src_v.at[i]], rows_v.at[b],
                              gsem.at[b]).wait()

    def scat(i, b):
        pltpu.async_copy(rows_v.at[b], acc.at[dst_v.at[i]], ssem.at[b],
                         add=True)

    def scat_wait(i, b):
        pltpu.make_async_copy(rows_v.at[b], acc.at[dst_v.at[i]],
                              ssem.at[b]).wait()

    for b in range(NBUF):
        gather(b, b)

    def body(j, carry):
        i0 = j * NBUF
        for b in range(NBUF):
            gather_wait(i0 + b, b)
            scat(i0 + b, b)
        for b in range(NBUF):
            scat_wait(i0 + b, b)
            gather(i0 + NBUF + b, b)
        return carry

    lax.fori_loop(0, njb - 1, body, 0)
    i0 = (njb - 1) * NBUF
    for b in range(NBUF):
        gather_wait(i0 + b, b)
        scat(i0 + b, b)
    for b in range(NBUF):
        scat_wait(i0 + b, b)


def _rsqrt16(d):
    i = lax.bitcast_convert_type(d, jnp.int32)
    y = lax.bitcast_convert_type(jnp.int32(0x5F3759DF) - (i >> 1),
                                 jnp.float32)
    for _ in range(3):
        y = y * (1.5 - 0.5 * d * y * y)
    return y


@functools.lru_cache(maxsize=None)
def _make_agg1(n_nodes, nctot):
    nch = nctot // NW
    rps = n_nodes // NS
    mesh = plsc.VectorSubcoreMesh(core_axis_name="c", subcore_axis_name="s")
    nf = jax.ShapeDtypeStruct((n_nodes, LANES), jnp.float32)

    @functools.partial(
        pl.kernel,
        mesh=mesh,
        compiler_params=pltpu.CompilerParams(use_tc_tiling_on_sc=False),
        out_type=(jax.ShapeDtypeStruct((NC, n_nodes, LANES), jnp.float32),
                  jax.ShapeDtypeStruct((NC, n_nodes, LANES), jnp.float32),
                  nf, nf, nf),
        scratch_types=[
            pltpu.VMEM((nch, BE), jnp.int32),
            pltpu.VMEM((nch, BE), jnp.int32),
            pltpu.VMEM((NBUF, BE, LANES), jnp.float32),
            pltpu.VMEM((rps, LANES), jnp.float32),
            pltpu.VMEM((rps, LANES), jnp.float32),
            pltpu.VMEM((rps, LANES), jnp.float32),
            pltpu.VMEM((rps, LANES), jnp.float32),
            pltpu.VMEM((rps, LANES), jnp.float32),
            pltpu.VMEM((LANES,), jnp.float32),
            pltpu.VMEM_SHARED((n_nodes + PADR, LANES), jnp.float32),
            pltpu.SemaphoreType.DMA((NBUF,)),
            pltpu.SemaphoreType.DMA((NBUF,)),
        ],
    )
    def agg1_kernel(xw_hbm, degp_hbm, eidx_hbm, b1_hbm, zeros_hbm,
                    s1p_out, xs_out, sb_out, dis_out, inv_out,
                    src_v, dst_v, rows_v, dga_v, xw_v, xs_v, sb_v, div_v,
                    b1_v, acc, gsem, ssem):
        c = lax.axis_index("c")
        s = lax.axis_index("s")
        wid = s * NC + c
        pltpu.sync_copy(zeros_hbm.at[pl.ds(s * rps, rps)],
                        acc.at[pl.ds(s * rps, rps)])
        pltpu.sync_copy(eidx_hbm.at[0, pl.ds(wid * nch, nch)], src_v)
        pltpu.sync_copy(eidx_hbm.at[1, pl.ds(wid * nch, nch)], dst_v)

        r0 = s * rps
        pltpu.sync_copy(degp_hbm.at[0, pl.ds(r0, rps)], dga_v)
        pltpu.sync_copy(xw_hbm.at[pl.ds(r0, rps)], xw_v)
        pltpu.sync_copy(b1_hbm, b1_v)
        pltpu.sync_copy(degp_hbm.at[1, pl.ds(r0, rps)], xs_v)

        def addb(r, carry):
            dga_v[r] = dga_v[r] + xs_v[r] + 1.0
            return carry

        lax.fori_loop(0, rps, addb, 0)
        b1vec = b1_v[...]

        def nbody(r, carry):
            d = dga_v[r]
            y = _rsqrt16(d)
            y2 = y * y
            w = xw_v[r]
            xs_v[r] = w * y
            sb_v[r] = w * y2 + b1vec
            dga_v[r] = y
            div_v[r] = y2
            return carry

        lax.fori_loop(0, rps, nbody, 0)
        pltpu.sync_copy(xs_v, xs_out.at[c, pl.ds(r0, rps)])

        @pl.when(c == 0)
        def _():
            pltpu.sync_copy(sb_v, sb_out.at[pl.ds(r0, rps)])
            pltpu.sync_copy(dga_v, dis_out.at[pl.ds(r0, rps)])
            pltpu.sync_copy(div_v, inv_out.at[pl.ds(r0, rps)])

        plsc.subcore_barrier()
        _edge_pipeline(xs_out.at[c], src_v, dst_v, rows_v, acc,
                       gsem, ssem, nch)
        plsc.subcore_barrier()
        pltpu.sync_copy(acc.at[pl.ds(s * rps, rps)],
                        s1p_out.at[c, pl.ds(s * rps, rps)])

    return agg1_kernel


@functools.lru_cache(maxsize=None)
def _make_agg(n_nodes, nctot):
    nch = nctot // NW
    rps = n_nodes // NS
    mesh = plsc.VectorSubcoreMesh(core_axis_name="c", subcore_axis_name="s")

    @functools.partial(
        pl.kernel,
        mesh=mesh,
        compiler_params=pltpu.CompilerParams(use_tc_tiling_on_sc=False),
        out_type=jax.ShapeDtypeStruct((NC, n_nodes, LANES), jnp.float32),
        scratch_types=[
            pltpu.VMEM((nch, BE), jnp.int32),
            pltpu.VMEM((nch, BE), jnp.int32),
            pltpu.VMEM((NBUF, BE, LANES), jnp.float32),
            pltpu.VMEM_SHARED((n_nodes + PADR, LANES), jnp.float32),
            pltpu.SemaphoreType.DMA((NBUF,)),
            pltpu.SemaphoreType.DMA((NBUF,)),
        ],
    )
    def agg_kernel(table_hbm, eidx_hbm, zeros_hbm, out_hbm,
                   src_v, dst_v, rows_v, acc, gsem, ssem):
        c = lax.axis_index("c")
        s = lax.axis_index("s")
        wid = s * NC + c
        pltpu.sync_copy(zeros_hbm.at[pl.ds(s * rps, rps)],
                        acc.at[pl.ds(s * rps, rps)])
        pltpu.sync_copy(eidx_hbm.at[0, pl.ds(wid * nch, nch)], src_v)
        pltpu.sync_copy(eidx_hbm.at[1, pl.ds(wid * nch, nch)], dst_v)
        plsc.subcore_barrier()
        _edge_pipeline(table_hbm, src_v, dst_v, rows_v, acc,
                       gsem, ssem, nch)
        plsc.subcore_barrier()
        pltpu.sync_copy(acc.at[pl.ds(s * rps, rps)],
                        out_hbm.at[c, pl.ds(s * rps, rps)])

    return agg_kernel


def _tc_mm1(x_ref, w1_ref, o_ref):
    o_ref[...] = jnp.dot(x_ref[...], w1_ref[...],
                         preferred_element_type=jnp.float32)


def _tc_mid(s1a_ref, s1b_ref, sb_ref, dis_ref, inv_ref, w2bd_ref, b2t_ref,
            hs_ref, self2_ref):
    h = jnp.maximum(dis_ref[...] * (s1a_ref[...] + s1b_ref[...])
                    + sb_ref[...], 0.0)
    hw = jnp.dot(h, w2bd_ref[...], preferred_element_type=jnp.float32)
    hs_ref[...] = hw * dis_ref[...]
    self2_ref[...] = hw * inv_ref[...] + b2t_ref[...]


def _tc_post(s2a_ref, s2b_ref, self2_ref, dis_ref, selp_ref, sumbd_ref,
             out_ref, *, d_out):
    o = dis_ref[...] * (s2a_ref[...] + s2b_ref[...]) + self2_ref[...]
    w = o
    for sh in (1, 2, 4, 8):
        w = jnp.maximum(w, jnp.roll(w, sh, axis=1))
    gmax = jnp.dot(w, selp_ref[...], preferred_element_type=jnp.float32)
    col = lax.broadcasted_iota(jnp.int32, o.shape, 1)
    e = jnp.where((col & (LANES - 1)) < d_out, jnp.exp(o - gmax), 0.0)
    ssum = jnp.dot(e, sumbd_ref[...], preferred_element_type=jnp.float32)
    out_ref[...] = o - gmax - jnp.log(ssum)


BN = 2000
BP = 250


def kernel(x, edge_index, W1, b1, W2, b2):
    n, d_in = x.shape
    d_hid = W1.shape[1]
    d_out = W2.shape[1]
    n_edges = edge_index.shape[1]
    assert d_hid == LANES and d_out <= LANES
    assert n % NS == 0 and n % BN == 0 and (n * LANES) % (BP * PACK) == 0
    npk = n * LANES // PACK

    nctot = -(-n_edges // (NW * BE * NBUF)) * NW * NBUF
    e_pad = nctot * BE - n_edges
    spread = jnp.arange(e_pad, dtype=jnp.int32)
    pad = jnp.concatenate(
        [(spread * 61 % n)[None, :],
         (n + spread % PADR)[None, :]], axis=0)
    eidx = jnp.concatenate([edge_index, pad], axis=1).reshape(2, nctot, BE)

    zeros = jnp.zeros((n, LANES), jnp.float32)
    ones = jnp.ones((BE, LANES), jnp.float32)
    w2p = jnp.zeros((LANES, LANES), jnp.float32).at[:d_hid, :d_out].set(W2)
    eye8 = jnp.eye(PACK // LANES, dtype=jnp.float32)
    w2bd = jnp.kron(eye8, w2p)
    b2t = jnp.tile(jnp.zeros((LANES,), jnp.float32).at[:d_out].set(b2),
                   PACK // LANES)
    ar = jnp.arange(PACK, dtype=jnp.int32)
    selp = (ar[:, None] == (ar[None, :] // LANES) * LANES + LANES - 1
            ).astype(jnp.float32)
    sumbd = jnp.kron(eye8, jnp.ones((LANES, LANES), jnp.float32))

    rowsP = pl.BlockSpec((npk, PACK), lambda: (0, 0))
    fullM = pl.BlockSpec((PACK, PACK), lambda: (0, 0))
    fullV = pl.BlockSpec((PACK,), lambda: (0,))
    pgrid = ()

    def P(a):
        return jnp.reshape(a, (npk, PACK))

    xw = pl.pallas_call(
        _tc_mm1, out_shape=jax.ShapeDtypeStruct((n, LANES), jnp.float32),
        grid=(n // BN,),
        in_specs=[pl.BlockSpec((BN, d_in), lambda i: (i, 0)),
                  pl.BlockSpec((d_in, LANES), lambda i: (0, 0))],
        out_specs=pl.BlockSpec((BN, LANES), lambda i: (i, 0)))(x, W1)

    degp = _make_deg(n, nctot)(eidx, zeros, ones)
    s1p, _, sb, dis, inv = _make_agg1(n, nctot)(xw, degp, eidx, b1, zeros)

    hs_p, self2_p = pl.pallas_call(
        _tc_mid,
        out_shape=[jax.ShapeDtypeStruct((npk, PACK), jnp.float32)] * 2,
        grid=pgrid,
        in_specs=[rowsP] * 5 + [fullM, fullV],
        out_specs=[rowsP] * 2,
    )(P(s1p[0]), P(s1p[1]), P(sb), P(dis), P(inv), w2bd, b2t)

    s2p = _make_agg(n, nctot)(jnp.reshape(hs_p, (n, LANES)), eidx, zeros)

    out_p = pl.pallas_call(
        functools.partial(_tc_post, d_out=d_out),
        out_shape=jax.ShapeDtypeStruct((npk, PACK), jnp.float32),
        grid=pgrid,
        in_specs=[rowsP] * 4 + [fullM, fullM],
        out_specs=rowsP,
    )(P(s2p[0]), P(s2p[1]), self2_p, P(dis), selp, sumbd)

    return jnp.reshape(out_p, (n, LANES))[:, :d_out]


# speedup vs baseline: 1.3484x; 1.3484x over previous
import functools

import jax
import jax.numpy as jnp
from jax import lax
from jax.experimental import pallas as pl
from jax.experimental.pallas import tpu as pltpu
from jax.experimental.pallas import tpu_sc as plsc

NC = 2
NS = 16
NW = NC * NS
LANES = 16
BE = 128
NBUF = 8
PADR = 512
PACK = 128


@functools.lru_cache(maxsize=None)
def _make_deg(n_nodes, nctot):
    nch = nctot // NW
    njb = nch // NBUF
    rps = n_nodes // NS
    mesh = plsc.VectorSubcoreMesh(core_axis_name="c", subcore_axis_name="s")

    @functools.partial(
        pl.kernel,
        mesh=mesh,
        compiler_params=pltpu.CompilerParams(use_tc_tiling_on_sc=False),
        out_type=jax.ShapeDtypeStruct((NC, n_nodes, LANES), jnp.float32),
        scratch_types=[
            pltpu.VMEM((nch, BE), jnp.int32),
            pltpu.VMEM((BE, LANES), jnp.float32),
            pltpu.VMEM_SHARED((n_nodes + PADR, LANES), jnp.float32),
            pltpu.SemaphoreType.DMA((NBUF,)),
        ],
    )
    def deg_kernel(eidx_hbm, zeros_hbm, ones_hbm, out_hbm,
                   dst_v, ones_v, acc, dsem):
        c = lax.axis_index("c")
        s = lax.axis_index("s")
        wid = s * NC + c
        pltpu.sync_copy(zeros_hbm.at[pl.ds(s * rps, rps)],
                        acc.at[pl.ds(s * rps, rps)])
        pltpu.sync_copy(ones_hbm, ones_v)
        pltpu.sync_copy(eidx_hbm.at[1, pl.ds(wid * nch, nch)], dst_v)
        plsc.subcore_barrier()

        def scat(i, b):
            pltpu.async_copy(ones_v, acc.at[dst_v.at[i]], dsem.at[b],
                             add=True)

        def scat_wait(i, b):
            pltpu.make_async_copy(ones_v, acc.at[dst_v.at[i]],
                                  dsem.at[b]).wait()

        for b in range(NBUF):
            scat(b, b)

        def body(j, carry):
            i0 = j * NBUF
            for b in range(NBUF):
                scat_wait(i0 + b, b)
                scat(i0 + NBUF + b, b)
            return carry

        lax.fori_loop(0, njb - 1, body, 0)
        i0 = (njb - 1) * NBUF
        for b in range(NBUF):
            scat_wait(i0 + b, b)
        plsc.subcore_barrier()
        pltpu.sync_copy(acc.at[pl.ds(s * rps, rps)],
                        out_hbm.at[c, pl.ds(s * rps, rps)])

    return deg_kernel


def _edge_pipeline(table, src_v, dst_v, rows_v, acc, gsem, ssem, nch):
    njb = nch // NBUF

    def gather(i, b):
        pltpu.async_copy(table.at[src_v.at[i]], rows_v.at[b], gsem.at[b])

    def gather_wait(i, b):
        pltpu.make_async_copy(table.at[src_v.at[i]], rows_v.at[b],
                              gsem.at[b]).wait()

    def scat(i, b):
        pltpu.async_copy(rows_v.at[b], acc.at[dst_v.at[i]], ssem.at[b],
                         add=True)

    def scat_wait(i, b):
        pltpu.make_async_copy(rows_v.at[b], acc.at[dst_v.at[i]],
                              ssem.at[b]).wait()

    for b in range(NBUF):
        gather(b, b)

    def body(j, carry):
        i0 = j * NBUF
        for b in range(NBUF):
            gather_wait(i0 + b, b)
            scat(i0 + b, b)
        for b in range(NBUF):
            scat_wait(i0 + b, b)
            gather(i0 + NBUF + b, b)
        return carry

    lax.fori_loop(0, njb - 1, body, 0)
    i0 = (njb - 1) * NBUF
    for b in range(NBUF):
        gather_wait(i0 + b, b)
        scat(i0 + b, b)
    for b in range(NBUF):
        scat_wait(i0 + b, b)


def _rsqrt16(d):
    i = lax.bitcast_convert_type(d, jnp.int32)
    y = lax.bitcast_convert_type(jnp.int32(0x5F3759DF) - (i >> 1),
                                 jnp.float32)
    for _ in range(3):
        y = y * (1.5 - 0.5 * d * y * y)
    return y


@functools.lru_cache(maxsize=None)
def _make_agg1(n_nodes, nctot):
    nch = nctot // NW
    rps = n_nodes // NS
    mesh = plsc.VectorSubcoreMesh(core_axis_name="c", subcore_axis_name="s")
    nf = jax.ShapeDtypeStruct((n_nodes, LANES), jnp.float32)

    @functools.partial(
        pl.kernel,
        mesh=mesh,
        compiler_params=pltpu.CompilerParams(use_tc_tiling_on_sc=False),
        out_type=(nf, nf,
                  jax.ShapeDtypeStruct((NC, n_nodes, LANES), jnp.float32),
                  nf, nf, nf),
        scratch_types=[
            pltpu.VMEM((nch, BE), jnp.int32),
            pltpu.VMEM((nch, BE), jnp.int32),
            pltpu.VMEM((NBUF, BE, LANES), jnp.float32),
            pltpu.VMEM((rps, LANES), jnp.float32),
            pltpu.VMEM((rps, LANES), jnp.float32),
            pltpu.VMEM((rps, LANES), jnp.float32),
            pltpu.VMEM((rps, LANES), jnp.float32),
            pltpu.VMEM((rps, LANES), jnp.float32),
            pltpu.VMEM((LANES,), jnp.float32),
            pltpu.VMEM_SHARED((n_nodes + PADR, LANES), jnp.float32),
            pltpu.SemaphoreType.DMA((NBUF,)),
            pltpu.SemaphoreType.DMA((NBUF,)),
        ],
    )
    def agg1_kernel(xw_hbm, degp_hbm, eidx_hbm, b1_hbm, zeros_hbm,
                    s1a_out, s1b_out, xs_out, sb_out, dis_out, inv_out,
                    src_v, dst_v, rows_v, dga_v, xw_v, xs_v, sb_v, div_v,
                    b1_v, acc, gsem, ssem):
        c = lax.axis_index("c")
        s = lax.axis_index("s")
        wid = s * NC + c
        pltpu.sync_copy(zeros_hbm.at[pl.ds(s * rps, rps)],
                        acc.at[pl.ds(s * rps, rps)])
        pltpu.sync_copy(eidx_hbm.at[0, pl.ds(wid * nch, nch)], src_v)
        pltpu.sync_copy(eidx_hbm.at[1, pl.ds(wid * nch, nch)], dst_v)

        r0 = s * rps
        pltpu.sync_copy(degp_hbm.at[0, pl.ds(r0, rps)], dga_v)
        pltpu.sync_copy(xw_hbm.at[pl.ds(r0, rps)], xw_v)
        pltpu.sync_copy(b1_hbm, b1_v)
        pltpu.sync_copy(degp_hbm.at[1, pl.ds(r0, rps)], xs_v)
        b1vec = b1_v[...]

        def nbody(r, carry):
            d = dga_v[r] + xs_v[r] + 1.0
            y = _rsqrt16(d)
            y2 = y * y
            w = xw_v[r]
            xs_v[r] = w * y
            sb_v[r] = w * y2 + b1vec
            dga_v[r] = y
            div_v[r] = y2
            return carry

        lax.fori_loop(0, rps, nbody, 0)
        pltpu.sync_copy(xs_v, xs_out.at[c, pl.ds(r0, rps)])

        @pl.when(c == 0)
        def _():
            pltpu.sync_copy(sb_v, sb_out.at[pl.ds(r0, rps)])
            pltpu.sync_copy(dga_v, dis_out.at[pl.ds(r0, rps)])
            pltpu.sync_copy(div_v, inv_out.at[pl.ds(r0, rps)])

        plsc.subcore_barrier()
        _edge_pipeline(xs_out.at[c], src_v, dst_v, rows_v, acc,
                       gsem, ssem, nch)
        plsc.subcore_barrier()

        @pl.when(c == 0)
        def _():
            pltpu.sync_copy(acc.at[pl.ds(s * rps, rps)],
                            s1a_out.at[pl.ds(s * rps, rps)])

        @pl.when(c == 1)
        def _():
            pltpu.sync_copy(acc.at[pl.ds(s * rps, rps)],
                            s1b_out.at[pl.ds(s * rps, rps)])

    return agg1_kernel


@functools.lru_cache(maxsize=None)
def _make_agg(n_nodes, nctot):
    nch = nctot // NW
    rps = n_nodes // NS
    mesh = plsc.VectorSubcoreMesh(core_axis_name="c", subcore_axis_name="s")

    @functools.partial(
        pl.kernel,
        mesh=mesh,
        compiler_params=pltpu.CompilerParams(use_tc_tiling_on_sc=False),
        out_type=(jax.ShapeDtypeStruct((n_nodes, LANES), jnp.float32),
                  jax.ShapeDtypeStruct((n_nodes, LANES), jnp.float32)),
        scratch_types=[
            pltpu.VMEM((nch, BE), jnp.int32),
            pltpu.VMEM((nch, BE), jnp.int32),
            pltpu.VMEM((NBUF, BE, LANES), jnp.float32),
            pltpu.VMEM_SHARED((n_nodes + PADR, LANES), jnp.float32),
            pltpu.SemaphoreType.DMA((NBUF,)),
            pltpu.SemaphoreType.DMA((NBUF,)),
        ],
    )
    def agg_kernel(table_hbm, eidx_hbm, zeros_hbm, outa_hbm, outb_hbm,
                   src_v, dst_v, rows_v, acc, gsem, ssem):
        c = lax.axis_index("c")
        s = lax.axis_index("s")
        wid = s * NC + c
        pltpu.sync_copy(zeros_hbm.at[pl.ds(s * rps, rps)],
                        acc.at[pl.ds(s * rps, rps)])
        pltpu.sync_copy(eidx_hbm.at[0, pl.ds(wid * nch, nch)], src_v)
        pltpu.sync_copy(eidx_hbm.at[1, pl.ds(wid * nch, nch)], dst_v)
        plsc.subcore_barrier()
        _edge_pipeline(table_hbm, src_v, dst_v, rows_v, acc,
                       gsem, ssem, nch)
        plsc.subcore_barrier()

        @pl.when(c == 0)
        def _():
            pltpu.sync_copy(acc.at[pl.ds(s * rps, rps)],
                            outa_hbm.at[pl.ds(s * rps, rps)])

        @pl.when(c == 1)
        def _():
            pltpu.sync_copy(acc.at[pl.ds(s * rps, rps)],
                            outb_hbm.at[pl.ds(s * rps, rps)])

    return agg_kernel


def _tc_mm1(x_ref, w1_ref, o_ref):
    o_ref[...] = jnp.dot(x_ref[...], w1_ref[...],
                         preferred_element_type=jnp.float32)


def _tc_mid(s1a_ref, s1b_ref, sb_ref, dis_ref, inv_ref, w2bd_ref, b2t_ref,
            hs_ref, self2_ref):
    h = jnp.maximum(dis_ref[...] * (s1a_ref[...] + s1b_ref[...])
                    + sb_ref[...], 0.0)
    hw = jnp.dot(h, w2bd_ref[...], preferred_element_type=jnp.float32)
    hs_ref[...] = hw * dis_ref[...]
    self2_ref[...] = hw * inv_ref[...] + b2t_ref[...]


def _tc_post(s2a_ref, s2b_ref, self2_ref, dis_ref, selp_ref, sumbd_ref,
             out_ref, *, d_out):
    o = dis_ref[...] * (s2a_ref[...] + s2b_ref[...]) + self2_ref[...]
    w = o
    for sh in (1, 2, 4, 8):
        w = jnp.maximum(w, jnp.roll(w, sh, axis=1))
    gmax = jnp.dot(w, selp_ref[...], preferred_element_type=jnp.float32)
    col = lax.broadcasted_iota(jnp.int32, o.shape, 1)
    e = jnp.where((col & (LANES - 1)) < d_out, jnp.exp(o - gmax), 0.0)
    ssum = jnp.dot(e, sumbd_ref[...], preferred_element_type=jnp.float32)
    out_ref[...] = o - gmax - jnp.log(ssum)


BN = 2000
BP = 250


def kernel(x, edge_index, W1, b1, W2, b2):
    n, d_in = x.shape
    d_hid = W1.shape[1]
    d_out = W2.shape[1]
    n_edges = edge_index.shape[1]
    assert d_hid == LANES and d_out <= LANES
    assert n % NS == 0 and n % BN == 0 and (n * LANES) % (BP * PACK) == 0
    npk = n * LANES // PACK

    nctot = -(-n_edges // (NW * BE * NBUF)) * NW * NBUF
    e_pad = nctot * BE - n_edges
    spread = jnp.arange(e_pad, dtype=jnp.int32)
    pad = jnp.concatenate(
        [(spread * 61 % n)[None, :],
         (n + spread % PADR)[None, :]], axis=0)
    eidx = jnp.concatenate([edge_index, pad], axis=1).reshape(2, nctot, BE)

    zeros = jnp.zeros((n, LANES), jnp.float32)
    ones = jnp.ones((BE, LANES), jnp.float32)
    w2p = jnp.zeros((LANES, LANES), jnp.float32).at[:d_hid, :d_out].set(W2)
    eye8 = jnp.eye(PACK // LANES, dtype=jnp.float32)
    w2bd = jnp.kron(eye8, w2p)
    b2t = jnp.tile(jnp.zeros((LANES,), jnp.float32).at[:d_out].set(b2),
                   PACK // LANES)
    ar = jnp.arange(PACK, dtype=jnp.int32)
    selp = (ar[:, None] == (ar[None, :] // LANES) * LANES + LANES - 1
            ).astype(jnp.float32)
    sumbd = jnp.kron(eye8, jnp.ones((LANES, LANES), jnp.float32))

    rowsP = pl.BlockSpec((npk, PACK), lambda: (0, 0))
    fullM = pl.BlockSpec((PACK, PACK), lambda: (0, 0))
    fullV = pl.BlockSpec((PACK,), lambda: (0,))
    pgrid = ()

    def P(a):
        return jnp.reshape(a, (npk, PACK))

    xw = pl.pallas_call(
        _tc_mm1, out_shape=jax.ShapeDtypeStruct((n, LANES), jnp.float32),
        grid=(n // BN,),
        in_specs=[pl.BlockSpec((BN, d_in), lambda i: (i, 0)),
                  pl.BlockSpec((d_in, LANES), lambda i: (0, 0))],
        out_specs=pl.BlockSpec((BN, LANES), lambda i: (i, 0)))(x, W1)

    degp = _make_deg(n, nctot)(eidx, zeros, ones)
    s1a, s1b, _, sb, dis, inv = _make_agg1(n, nctot)(xw, degp, eidx, b1,
                                                     zeros)

    hs_p, self2_p = pl.pallas_call(
        _tc_mid,
        out_shape=[jax.ShapeDtypeStruct((npk, PACK), jnp.float32)] * 2,
        grid=pgrid,
        in_specs=[rowsP] * 5 + [fullM, fullV],
        out_specs=[rowsP] * 2,
    )(P(s1a), P(s1b), P(sb), P(dis), P(inv), w2bd, b2t)

    s2a, s2b = _make_agg(n, nctot)(jnp.reshape(hs_p, (n, LANES)), eidx,
                                   zeros)

    out_p = pl.pallas_call(
        functools.partial(_tc_post, d_out=d_out),
        out_shape=jax.ShapeDtypeStruct((npk, PACK), jnp.float32),
        grid=pgrid,
        in_specs=[rowsP] * 4 + [fullM, fullM],
        out_specs=rowsP,
    )(P(s2a), P(s2b), self2_p, P(dis), selp, sumbd)

    return jnp.reshape(out_p, (n, LANES))[:, :d_out]
